# baseline (speedup 1.0000x reference)
import functools

import jax
import jax.numpy as jnp
from jax import lax
from jax.experimental import pallas as pl
from jax.experimental.pallas import tpu as pltpu
from jax.experimental.pallas import tpu_sc as plsc

_N = 10000
_E = 320000
_NW = 32
_G = 80
_EPW = _E // _NW
_NG = _EPW // _G
_RPT = _N // 16

_mesh = plsc.VectorSubcoreMesh(core_axis_name="c", subcore_axis_name="s")


_BIDX = tuple(range(16))


def _bcast_lane(v, e):
    return lax.gather(
        v, jnp.full((16, 1), e, jnp.int32),
        lax.GatherDimensionNumbers(offset_dims=(), collapsed_slice_dims=(0,),
                                   start_index_map=(0,)),
        (1,), mode=lax.GatherScatterMode.PROMISE_IN_BOUNDS)


def _sc_degree_body(ei_hbm, w_hbm, zero_hbm, out_hbm, dst_v, w_v, wrows, acc):
    c = lax.axis_index("c")
    s = lax.axis_index("s")
    wid = c * 16 + s
    pltpu.sync_copy(ei_hbm.at[1, wid], dst_v)
    pltpu.sync_copy(w_hbm.at[wid], w_v)
    pltpu.sync_copy(zero_hbm.at[pl.ds(s * _RPT, _RPT)],
                    acc.at[pl.ds(s * _RPT, _RPT)])
    plsc.subcore_barrier()

    def body(j, carry):
        for gi in range(_G // 16):
            w16 = w_v[j, pl.ds(gi * 16, 16)]
            for e in range(16):
                wrows[gi * 16 + e, :] = _bcast_lane(w16, e)
        pltpu.sync_copy(wrows, acc.at[dst_v.at[j]], add=True)
        return carry

    lax.fori_loop(0, _NG, body, 0)
    plsc.subcore_barrier()
    pltpu.sync_copy(acc.at[pl.ds(s * _RPT, _RPT)],
                    out_hbm.at[c, pl.ds(s * _RPT, _RPT)])


_sc_degree = functools.partial(
    pl.kernel,
    out_type=jax.ShapeDtypeStruct((2, _N, 16), jnp.float32),
    mesh=_mesh,
    compiler_params=pltpu.CompilerParams(use_tc_tiling_on_sc=False, needs_layout_passes=False),
    scratch_types=[
        pltpu.VMEM((_NG, _G), jnp.int32),
        pltpu.VMEM((_NG, _G), jnp.float32),
        pltpu.VMEM((_G, 16), jnp.float32),
        pltpu.VMEM_SHARED((_N, 16), jnp.float32),
    ],
)(_sc_degree_body)


def _make_sc_layer_body(F):
    def _lay(g_hbm, ei_hbm, w_hbm, zero_hbm, out_hbm,
             src_v, dst_v, w_v, gr0, gr1, sr0, sr1, acc,
             gsem0, gsem1, ssem0, ssem1):
        c = lax.axis_index("c")
        s = lax.axis_index("s")
        wid = c * 16 + s
        pltpu.sync_copy(ei_hbm.at[0, wid], src_v)
        pltpu.sync_copy(ei_hbm.at[1, wid], dst_v)
        pltpu.sync_copy(w_hbm.at[wid], w_v)
        pltpu.sync_copy(zero_hbm.at[pl.ds(s * _RPT, _RPT)],
                        acc.at[pl.ds(s * _RPT, _RPT)])
        plsc.subcore_barrier()
        gr = (gr0, gr1)
        sr = (sr0, sr1)
        gsem = (gsem0, gsem1)
        ssem = (ssem0, ssem1)

        def scale(j, grv, srv):
            for gi in range(_G // 16):
                w16 = w_v[j, pl.ds(gi * 16, 16)]
                for e in range(16):
                    i = gi * 16 + e
                    wb = _bcast_lane(w16, e)
                    for h in range(F // 16):
                        sl = pl.ds(16 * h, 16)
                        srv[i, sl] = grv[i, sl] * wb

        def gather_start(j, p):
            pltpu.async_copy(g_hbm.at[src_v.at[j]], gr[p], gsem[p])

        def gather_wait(p):
            pltpu.make_async_copy(g_hbm.at[pl.ds(0, _G)], gr[p], gsem[p]).wait()

        def scatter_start(j, p):
            pltpu.async_copy(sr[p], acc.at[dst_v.at[j]], ssem[p], add=True)

        def scatter_wait(p):
            pltpu.make_async_copy(g_hbm.at[pl.ds(0, _G)], sr[p], ssem[p]).wait()

        gather_start(0, 0)
        gather_start(1, 1)
        for p in range(2):
            gather_wait(p)
            scale(p, gr[p], sr[p])
            scatter_start(p, p)
            gather_start(p + 2, p)

        def body(k, carry):
            for p in range(2):
                j = 2 * k + p
                gather_wait(p)
                scatter_wait(p)
                scale(j, gr[p], sr[p])
                scatter_start(j, p)

                @pl.when(j + 2 < _NG)
                def _():
                    gather_start(j + 2, p)

            return carry

        lax.fori_loop(1, 62, body, 0)
        gather_wait(0)
        scatter_wait(0)
        scale(_NG - 1, gr0, sr0)
        scatter_start(_NG - 1, 0)
        scatter_wait(0)
        scatter_wait(1)

        plsc.subcore_barrier()
        pltpu.sync_copy(acc.at[pl.ds(s * _RPT, _RPT)],
                        out_hbm.at[c, pl.ds(s * _RPT, _RPT)])

    return _lay


def _make_sc_layer(F):
    return functools.partial(
        pl.kernel,
        out_type=jax.ShapeDtypeStruct((2, _N, F), jnp.float32),
        mesh=_mesh,
        compiler_params=pltpu.CompilerParams(use_tc_tiling_on_sc=False, needs_layout_passes=False),
        scratch_types=[
            pltpu.VMEM((_NG, _G), jnp.int32),
            pltpu.VMEM((_NG, _G), jnp.int32),
            pltpu.VMEM((_NG, _G), jnp.float32),
            pltpu.VMEM((_G, F), jnp.float32),
            pltpu.VMEM((_G, F), jnp.float32),
            pltpu.VMEM((_G, F), jnp.float32),
            pltpu.VMEM((_G, F), jnp.float32),
            pltpu.VMEM_SHARED((_N, F), jnp.float32),
            pltpu.SemaphoreType.DMA,
            pltpu.SemaphoreType.DMA,
            pltpu.SemaphoreType.DMA,
            pltpu.SemaphoreType.DMA,
        ],
    )(_make_sc_layer_body(F))


_sc_layer16 = _make_sc_layer(16)
_sc_layer32 = _make_sc_layer(32)


def _tc_head(degp, x, Wg1):
    def body(degp_ref, x_ref, w_ref, dinv_ref, g1_ref):
        deg = degp_ref[0] + degp_ref[1] + 1.0
        dinv = lax.rsqrt(deg)
        dinv_ref[...] = dinv
        hw = jnp.dot(x_ref[...], w_ref[...], preferred_element_type=jnp.float32)
        g1_ref[...] = hw * dinv

    blk = 1000
    return pl.pallas_call(
        body,
        grid=(_N // blk,),
        in_specs=[pl.BlockSpec((2, blk, 16), lambda i: (0, i, 0)),
                  pl.BlockSpec((blk, 128), lambda i: (i, 0)),
                  pl.BlockSpec((128, 16), lambda i: (0, 0))],
        out_specs=[pl.BlockSpec((blk, 16), lambda i: (i, 0)),
                   pl.BlockSpec((blk, 16), lambda i: (i, 0))],
        out_shape=(jax.ShapeDtypeStruct((_N, 16), jnp.float32),
                   jax.ShapeDtypeStruct((_N, 16), jnp.float32)),
    )(degp, x, Wg1)


def _tc_mid(p, g, dinv, b, W, Fout):
    def body(p_ref, g_ref, dinv_ref, b_ref, W_ref, out_ref):
        dinv = dinv_ref[...]
        o = (p_ref[0] + p_ref[1] + g_ref[...]) * dinv + b_ref[...]
        h = jnp.tanh(o)
        dout = dinv if Fout == 16 else jnp.concatenate([dinv, dinv], axis=1)
        out_ref[...] = jnp.dot(h, W_ref[...],
                               preferred_element_type=jnp.float32) * dout

    blk = 1000
    return pl.pallas_call(
        body,
        grid=(_N // blk,),
        in_specs=[pl.BlockSpec((2, blk, 16), lambda i: (0, i, 0)),
                  pl.BlockSpec((blk, 16), lambda i: (i, 0)),
                  pl.BlockSpec((blk, 16), lambda i: (i, 0)),
                  pl.BlockSpec((1, 16), lambda i: (0, 0)),
                  pl.BlockSpec((16, Fout), lambda i: (0, 0))],
        out_specs=pl.BlockSpec((blk, Fout), lambda i: (i, 0)),
        out_shape=jax.ShapeDtypeStruct((_N, Fout), jnp.float32),
    )(p, g, dinv, b, W)


def _tc_tail(pos, p, g, dinv, bg3, W1, b1, W2, b2, W3, b3):
    def body(pos_ref, p_ref, g_ref, dinv_ref, bg3_ref,
             W1_ref, b1_ref, W2_ref, b2_ref, W3_ref, b3_ref, out_ref):
        rows = []
        for i in range(16):
            sl = pl.ds(pos_ref[i], 1)
            dv = dinv_ref[sl, :]
            dv32 = jnp.concatenate([dv, dv], axis=1)
            r = ((p_ref[0, sl, :] + p_ref[1, sl, :] + g_ref[sl, :])
                 * dv32 + bg3_ref[...])
            rows.append(r)
        flat = jnp.concatenate(rows, axis=1)
        h1 = jnp.tanh(jnp.dot(flat, W1_ref[...],
                              preferred_element_type=jnp.float32) + b1_ref[...])
        h2 = jnp.tanh(jnp.dot(h1, W2_ref[...],
                              preferred_element_type=jnp.float32) + b2_ref[...])
        out_ref[...] = jnp.dot(h2, W3_ref[...],
                               preferred_element_type=jnp.float32) + b3_ref[...]

    n_vmem = 10
    return pl.pallas_call(
        body,
        out_shape=jax.ShapeDtypeStruct((1, 16), jnp.float32),
        in_specs=[pl.BlockSpec(memory_space=pltpu.SMEM)]
                 + [pl.BlockSpec(memory_space=pltpu.VMEM)] * n_vmem,
    )(pos, p, g, dinv, bg3, W1, b1, W2, b2, W3, b3)


def kernel(x, edge_index, edge_weight, pos, Wg1, bg1, Wg2, bg2, Wg3, bg3,
           W1, b1, W2, b2, W3, b3):
    ei = edge_index.reshape(2, _NW, _NG, _G)
    w = edge_weight.reshape(_NW, _NG, _G)
    z16 = jnp.zeros((_N, 16), jnp.float32)
    z32 = jnp.zeros((_N, 32), jnp.float32)

    degp = _sc_degree(ei, w, z16)
    dinv, g1 = _tc_head(degp, x, Wg1)
    p1 = _sc_layer16(g1, ei, w, z16)
    g2 = _tc_mid(p1, g1, dinv, bg1.reshape(1, 16), Wg2, 16)
    p2 = _sc_layer16(g2, ei, w, z16)
    g3 = _tc_mid(p2, g2, dinv, bg2.reshape(1, 16), Wg3, 32)
    p3 = _sc_layer32(g3, ei, w, z32)
    return _tc_tail(pos, p3, g3, dinv, bg3.reshape(1, 32),
                    W1, b1.reshape(1, 128), W2, b2.reshape(1, 128),
                    W3, b3.reshape(1, 16))


# speedup vs baseline: 1.0193x; 1.0193x over previous
import functools

import jax
import jax.numpy as jnp
from jax import lax
from jax.experimental import pallas as pl
from jax.experimental.pallas import tpu as pltpu
from jax.experimental.pallas import tpu_sc as plsc

_N = 10000
_E = 320000
_NW = 32
_G = 80
_EPW = _E // _NW
_NG = _EPW // _G
_RPT = _N // 16

_mesh = plsc.VectorSubcoreMesh(core_axis_name="c", subcore_axis_name="s")


_BIDX = tuple(range(16))


def _bcast_lane(v, e):
    return lax.gather(
        v, jnp.full((16, 1), e, jnp.int32),
        lax.GatherDimensionNumbers(offset_dims=(), collapsed_slice_dims=(0,),
                                   start_index_map=(0,)),
        (1,), mode=lax.GatherScatterMode.PROMISE_IN_BOUNDS)


def _sc_degree_body(ei_hbm, w_hbm, zero_hbm, out_hbm, dst_v, w_v, wrows, acc):
    c = lax.axis_index("c")
    s = lax.axis_index("s")
    wid = c * 16 + s
    pltpu.sync_copy(ei_hbm.at[1, wid], dst_v)
    pltpu.sync_copy(w_hbm.at[wid], w_v)
    pltpu.sync_copy(zero_hbm.at[pl.ds(s * _RPT, _RPT)],
                    acc.at[pl.ds(s * _RPT, _RPT)])
    plsc.subcore_barrier()

    def body(j, carry):
        for gi in range(_G // 16):
            w16 = w_v[j, pl.ds(gi * 16, 16)]
            for e in range(16):
                wrows[gi * 16 + e, :] = _bcast_lane(w16, e)
        pltpu.sync_copy(wrows, acc.at[dst_v.at[j]], add=True)
        return carry

    lax.fori_loop(0, _NG, body, 0)
    plsc.subcore_barrier()
    pltpu.sync_copy(acc.at[pl.ds(s * _RPT, _RPT)],
                    out_hbm.at[c, pl.ds(s * _RPT, _RPT)])


_sc_degree = functools.partial(
    pl.kernel,
    out_type=jax.ShapeDtypeStruct((2, _N, 16), jnp.float32),
    mesh=_mesh,
    compiler_params=pltpu.CompilerParams(use_tc_tiling_on_sc=False, needs_layout_passes=False),
    scratch_types=[
        pltpu.VMEM((_NG, _G), jnp.int32),
        pltpu.VMEM((_NG, _G), jnp.float32),
        pltpu.VMEM((_G, 16), jnp.float32),
        pltpu.VMEM_SHARED((_N, 16), jnp.float32),
    ],
)(_sc_degree_body)


def _make_sc_layer_body(F):
    def _lay(g_hbm, ei_hbm, w_hbm, zero_hbm, out_hbm,
             src_v, dst_v, w_v, gr0, gr1, sr0, sr1, acc,
             gsem0, gsem1, ssem0, ssem1):
        c = lax.axis_index("c")
        s = lax.axis_index("s")
        wid = c * 16 + s
        pltpu.sync_copy(ei_hbm.at[0, wid], src_v)
        pltpu.sync_copy(ei_hbm.at[1, wid], dst_v)
        pltpu.sync_copy(w_hbm.at[wid], w_v)
        pltpu.sync_copy(zero_hbm.at[pl.ds(s * _RPT, _RPT)],
                        acc.at[pl.ds(s * _RPT, _RPT)])
        plsc.subcore_barrier()
        gr = (gr0, gr1)
        sr = (sr0, sr1)
        gsem = (gsem0, gsem1)
        ssem = (ssem0, ssem1)

        def scale(j, grv, srv):
            for gi in range(_G // 16):
                w16 = w_v[j, pl.ds(gi * 16, 16)]
                for e in range(16):
                    i = gi * 16 + e
                    wb = _bcast_lane(w16, e)
                    for h in range(F // 16):
                        sl = pl.ds(16 * h, 16)
                        srv[i, sl] = grv[i, sl] * wb

        def gather_start(j, p):
            pltpu.async_copy(g_hbm.at[src_v.at[j]], gr[p], gsem[p])

        def gather_wait(p):
            pltpu.make_async_copy(g_hbm.at[pl.ds(0, _G)], gr[p], gsem[p]).wait()

        def scatter_start(j, p):
            pltpu.async_copy(sr[p], acc.at[dst_v.at[j]], ssem[p], add=True)

        def scatter_wait(p):
            pltpu.make_async_copy(g_hbm.at[pl.ds(0, _G)], sr[p], ssem[p]).wait()

        gather_start(0, 0)
        gather_start(1, 1)
        for p in range(2):
            gather_wait(p)
            scale(p, gr[p], sr[p])
            scatter_start(p, p)
            gather_start(p + 2, p)

        def body(k, carry):
            for p in range(2):
                j = 2 * k + p
                gather_wait(p)
                scatter_wait(p)
                scale(j, gr[p], sr[p])
                scatter_start(j, p)

                @pl.when(j + 2 < _NG)
                def _():
                    gather_start(j + 2, p)

            return carry

        lax.fori_loop(1, 62, body, 0)
        gather_wait(0)
        scatter_wait(0)
        scale(_NG - 1, gr0, sr0)
        scatter_start(_NG - 1, 0)
        scatter_wait(0)
        scatter_wait(1)

        plsc.subcore_barrier()
        pltpu.sync_copy(acc.at[pl.ds(s * _RPT, _RPT)],
                        out_hbm.at[c, pl.ds(s * _RPT, _RPT)])

    return _lay


def _make_sc_layer(F):
    return functools.partial(
        pl.kernel,
        out_type=jax.ShapeDtypeStruct((2, _N, F), jnp.float32),
        mesh=_mesh,
        compiler_params=pltpu.CompilerParams(use_tc_tiling_on_sc=False, needs_layout_passes=False),
        scratch_types=[
            pltpu.VMEM((_NG, _G), jnp.int32),
            pltpu.VMEM((_NG, _G), jnp.int32),
            pltpu.VMEM((_NG, _G), jnp.float32),
            pltpu.VMEM((_G, F), jnp.float32),
            pltpu.VMEM((_G, F), jnp.float32),
            pltpu.VMEM((_G, F), jnp.float32),
            pltpu.VMEM((_G, F), jnp.float32),
            pltpu.VMEM_SHARED((_N, F), jnp.float32),
            pltpu.SemaphoreType.DMA,
            pltpu.SemaphoreType.DMA,
            pltpu.SemaphoreType.DMA,
            pltpu.SemaphoreType.DMA,
        ],
    )(_make_sc_layer_body(F))


_sc_layer16 = _make_sc_layer(16)
_sc_layer32 = _make_sc_layer(32)


def _tc_head(degp, x, Wg1):
    def body(degp_ref, x_ref, w_ref, dinv_ref, g1_ref):
        deg = degp_ref[0] + degp_ref[1] + 1.0
        dinv = lax.rsqrt(deg)
        dinv_ref[...] = dinv
        hw = jnp.dot(x_ref[...], w_ref[...], preferred_element_type=jnp.float32)
        g1_ref[...] = hw * dinv

    return pl.pallas_call(
        body,
        out_shape=(jax.ShapeDtypeStruct((_N, 16), jnp.float32),
                   jax.ShapeDtypeStruct((_N, 16), jnp.float32)),
    )(degp, x, Wg1)


def _tc_mid(p, g, dinv, b, W, Fout):
    def body(p_ref, g_ref, dinv_ref, b_ref, W_ref, out_ref):
        dinv = dinv_ref[...]
        o = (p_ref[0] + p_ref[1] + g_ref[...]) * dinv + b_ref[...]
        h = jnp.tanh(o)
        dout = dinv if Fout == 16 else jnp.concatenate([dinv, dinv], axis=1)
        out_ref[...] = jnp.dot(h, W_ref[...],
                               preferred_element_type=jnp.float32) * dout

    return pl.pallas_call(
        body,
        out_shape=jax.ShapeDtypeStruct((_N, Fout), jnp.float32),
    )(p, g, dinv, b, W)


def _tc_tail(pos, p, g, dinv, bg3, W1, b1, W2, b2, W3, b3):
    def body(pos_ref, p_ref, g_ref, dinv_ref, bg3_ref,
             W1_ref, b1_ref, W2_ref, b2_ref, W3_ref, b3_ref, out_ref):
        rows = []
        for i in range(16):
            sl = pl.ds(pos_ref[i], 1)
            dv = dinv_ref[sl, :]
            dv32 = jnp.concatenate([dv, dv], axis=1)
            r = ((p_ref[0, sl, :] + p_ref[1, sl, :] + g_ref[sl, :])
                 * dv32 + bg3_ref[...])
            rows.append(r)
        flat = jnp.concatenate(rows, axis=1)
        h1 = jnp.tanh(jnp.dot(flat, W1_ref[...],
                              preferred_element_type=jnp.float32) + b1_ref[...])
        h2 = jnp.tanh(jnp.dot(h1, W2_ref[...],
                              preferred_element_type=jnp.float32) + b2_ref[...])
        out_ref[...] = jnp.dot(h2, W3_ref[...],
                               preferred_element_type=jnp.float32) + b3_ref[...]

    n_vmem = 10
    return pl.pallas_call(
        body,
        out_shape=jax.ShapeDtypeStruct((1, 16), jnp.float32),
        in_specs=[pl.BlockSpec(memory_space=pltpu.SMEM)]
                 + [pl.BlockSpec(memory_space=pltpu.VMEM)] * n_vmem,
    )(pos, p, g, dinv, bg3, W1, b1, W2, b2, W3, b3)


def kernel(x, edge_index, edge_weight, pos, Wg1, bg1, Wg2, bg2, Wg3, bg3,
           W1, b1, W2, b2, W3, b3):
    ei = edge_index.reshape(2, _NW, _NG, _G)
    w = edge_weight.reshape(_NW, _NG, _G)
    z16 = jnp.zeros((_N, 16), jnp.float32)
    z32 = jnp.zeros((_N, 32), jnp.float32)

    degp = _sc_degree(ei, w, z16)
    dinv, g1 = _tc_head(degp, x, Wg1)
    p1 = _sc_layer16(g1, ei, w, z16)
    g2 = _tc_mid(p1, g1, dinv, bg1.reshape(1, 16), Wg2, 16)
    p2 = _sc_layer16(g2, ei, w, z16)
    g3 = _tc_mid(p2, g2, dinv, bg2.reshape(1, 16), Wg3, 32)
    p3 = _sc_layer32(g3, ei, w, z32)
    return _tc_tail(pos, p3, g3, dinv, bg3.reshape(1, 32),
                    W1, b1.reshape(1, 128), W2, b2.reshape(1, 128),
                    W3, b3.reshape(1, 16))
